# baseline (speedup 1.0000x reference)
import functools

import jax
import jax.numpy as jnp
from jax import lax
from jax.experimental import pallas as pl
from jax.experimental.pallas import tpu as pltpu
from jax.experimental.pallas import tpu_sc as plsc

VOCAB = 1000000
D = 32
B = 16384
P = 10
N = 50
CTX = P + N
LOGIT_COLS = 64

VB = 7936
QB = VB // 4
TBLOCKS = -(-VOCAB // VB)
TROWS = TBLOCKS * VB

NC = 2
NS = 16
NW = NC * NS
B_PER_W = B // NW
NB = 32
CHUNKS = B_PER_W // NB
IDX_ROWS = NB * CTX // 128
ROW_PAD = 4


def _transpose_body(x_ref, o_ref):
  x = x_ref[:]
  eye = jnp.eye(D, dtype=jnp.float32)
  xt = lax.dot_general(x, eye, (((0,), (0,)), ((), ())),
                       preferred_element_type=jnp.float32)
  for j in range(4):
    o_ref[:, 32 * j:32 * (j + 1)] = xt[QB * j:QB * (j + 1), :]


def _to_linear_table(table):
  out = pl.pallas_call(
      _transpose_body,
      grid=(TBLOCKS,),
      in_specs=[pl.BlockSpec((D, VB), lambda i: (0, i))],
      out_specs=pl.BlockSpec((QB, 128), lambda i: (i, 0)),
      out_shape=jax.ShapeDtypeStruct((TBLOCKS * QB, 128), jnp.float32),
  )(table.T)
  return out.reshape(TROWS, D)


def _remap_rows(v):
  rem = v % VB
  return VB * (v // VB) + 4 * (rem % QB) + rem // QB


def _sc_body(in_tbl, out_tbl, labels, ctx_idx, logits, cidx_v, lidx_v,
             rows_v, ctr_v, log_v, sem):
  wid = lax.axis_index("s") * NC + lax.axis_index("c")
  iota16 = lax.iota(jnp.int32, 16)
  col_idx = [jnp.full((16,), d, jnp.int32) for d in range(D)]

  def chunk_body(c, carry):
    blk = wid * CHUNKS + c
    b_base = pl.multiple_of(blk * NB, NB)

    pltpu.sync_copy(ctx_idx.at[blk], cidx_v)
    pltpu.sync_copy(labels.at[pl.ds(b_base, NB)], lidx_v)

    copies = [pltpu.async_copy(in_tbl.at[lidx_v], ctr_v, sem)]
    for j in range(IDX_ROWS):
      copies.append(
          pltpu.async_copy(out_tbl.at[cidx_v.at[j]],
                           rows_v.at[pl.ds(j * 128, 128)], sem))
    for cp in copies:
      cp.wait()

    def b_body(bl, inner):
      base_pair = bl * CTX
      row0 = iota16 + base_pair
      row_idx = [row0 + g * 16 for g in range(LOGIT_COLS // 16)]
      accs = [jnp.zeros((16,), jnp.float32) for _ in range(LOGIT_COLS // 16)]
      ctr_half = [ctr_v[bl, pl.ds(0, 16)], ctr_v[bl, pl.ds(16, 16)]]
      for d in range(D):
        s = ctr_half[d // 16][d % 16]
        for g in range(LOGIT_COLS // 16):
          v = plsc.load_gather(rows_v, [row_idx[g], col_idx[d]])
          accs[g] = accs[g] + v * s
      for g in range(LOGIT_COLS // 16):
        log_v[bl, pl.ds(g * 16, 16)] = accs[g]
      return inner

    lax.fori_loop(0, NB, b_body, 0)
    pltpu.sync_copy(log_v, logits.at[pl.ds(b_base, NB)])
    return carry

  lax.fori_loop(0, CHUNKS, chunk_body, 0)


def _sc_logits(in_tbl, out_tbl, labels, ctx_rows):
  mesh = plsc.VectorSubcoreMesh(core_axis_name="c", subcore_axis_name="s")
  kfn = pl.kernel(
      _sc_body,
      out_type=jax.ShapeDtypeStruct((B, LOGIT_COLS), jnp.float32),
      mesh=mesh,
      compiler_params=pltpu.CompilerParams(
          needs_layout_passes=False, use_tc_tiling_on_sc=False),
      scratch_types=[
          pltpu.VMEM((IDX_ROWS, 128), jnp.int32),
          pltpu.VMEM((NB,), jnp.int32),
          pltpu.VMEM((NB * CTX + ROW_PAD, D), jnp.float32),
          pltpu.VMEM((NB, D), jnp.float32),
          pltpu.VMEM((NB, LOGIT_COLS), jnp.float32),
          pltpu.SemaphoreType.DMA,
      ],
  )
  return kfn(in_tbl, out_tbl, labels, ctx_rows)


def _tc_body(m_ref, o_ref):
  x = m_ref[:]
  col = lax.broadcasted_iota(jnp.int32, x.shape, 1)
  sign = jnp.where(col < P, 1.0, -1.0).astype(jnp.float32)
  w = jnp.where(col < P, 1.0 / P,
                jnp.where(col < CTX, 1.0 / N, 0.0)).astype(jnp.float32)
  z = x * sign
  ls = jnp.minimum(z, 0.0) - jnp.log1p(jnp.exp(-jnp.abs(z)))
  o_ref[:] = -jnp.sum(ls * w, axis=1)


def kernel(input_labels, pos_labels, neg_labels, in_embed, out_embed):
  labels = _remap_rows(input_labels.astype(jnp.int32))
  ctx = _remap_rows(jnp.concatenate(
      [pos_labels.astype(jnp.int32), neg_labels.astype(jnp.int32)], axis=1))
  ctx_rows = ctx.reshape(NW * CHUNKS, IDX_ROWS, 128)

  in_tbl = _to_linear_table(in_embed)
  out_tbl = _to_linear_table(out_embed)

  logits = _sc_logits(in_tbl, out_tbl, labels, ctx_rows)

  return pl.pallas_call(
      _tc_body,
      out_shape=jax.ShapeDtypeStruct((B,), jnp.float32),
  )(logits)


# speedup vs baseline: 1.3054x; 1.3054x over previous
import functools

import jax
import jax.numpy as jnp
from jax import lax
from jax.experimental import pallas as pl
from jax.experimental.pallas import tpu as pltpu
from jax.experimental.pallas import tpu_sc as plsc

VOCAB = 1000000
D = 32
B = 16384
P = 10
N = 50
CTX = P + N
LOGIT_COLS = 64

VB = 8192
TBLOCKS = -(-VOCAB // VB)
TROWS = TBLOCKS * VB

NC = 2
NS = 16
NW = NC * NS
B_PER_W = B // NW
NB = 32
CHUNKS = B_PER_W // NB
IDX_ROWS = NB * CTX // 128
ROW_PAD = 4


def _transpose_body(x_ref, o_ref):
  x = x_ref[:]
  xx = jnp.concatenate(
      [x[:, 1024 * k:1024 * (k + 1)] for k in range(8)], axis=0)
  eye = jnp.eye(256, dtype=jnp.float32)
  y = lax.dot_general(xx, eye, (((0,), (0,)), ((), ())),
                      preferred_element_type=jnp.float32)
  o_ref[0:1024, :] = y[:, 0:128]
  o_ref[1024:2048, :] = y[:, 128:256]


def _to_linear_table(table):
  out = pl.pallas_call(
      _transpose_body,
      grid=(TBLOCKS,),
      in_specs=[pl.BlockSpec((D, VB), lambda i: (0, i))],
      out_specs=pl.BlockSpec((2048, 128), lambda i: (i, 0)),
      out_shape=jax.ShapeDtypeStruct((TBLOCKS * 2048, 128), jnp.float32),
  )(table.T)
  return out.reshape(TROWS, D)


def _remap_rows(v):
  return (v & ~8191) | (v & 4096) | ((v & 1023) << 2) | ((v >> 10) & 3)


def _sc_body(in_tbl, out_tbl, labels, ctx_idx, logits, cidx_v, lidx_v,
             rows_v, ctr_v, log_v, sem):
  wid = lax.axis_index("s") * NC + lax.axis_index("c")
  iota16 = lax.iota(jnp.int32, 16)
  col_idx = [jnp.full((16,), d, jnp.int32) for d in range(D)]

  def chunk_body(c, carry):
    blk = wid * CHUNKS + c
    b_base = pl.multiple_of(blk * NB, NB)

    pltpu.sync_copy(ctx_idx.at[blk], cidx_v)
    pltpu.sync_copy(labels.at[pl.ds(b_base, NB)], lidx_v)

    copies = [pltpu.async_copy(in_tbl.at[lidx_v], ctr_v, sem)]
    for j in range(IDX_ROWS):
      copies.append(
          pltpu.async_copy(out_tbl.at[cidx_v.at[j]],
                           rows_v.at[pl.ds(j * 128, 128)], sem))
    for cp in copies:
      cp.wait()

    def b_body(bl, inner):
      base_pair = bl * CTX
      row0 = iota16 + base_pair
      row_idx = [row0 + g * 16 for g in range(LOGIT_COLS // 16)]
      accs = [jnp.zeros((16,), jnp.float32) for _ in range(LOGIT_COLS // 16)]
      ctr_half = [ctr_v[bl, pl.ds(0, 16)], ctr_v[bl, pl.ds(16, 16)]]
      for d in range(D):
        s = ctr_half[d // 16][d % 16]
        for g in range(LOGIT_COLS // 16):
          v = plsc.load_gather(rows_v, [row_idx[g], col_idx[d]])
          accs[g] = accs[g] + v * s
      for g in range(LOGIT_COLS // 16):
        log_v[bl, pl.ds(g * 16, 16)] = accs[g]
      return inner

    lax.fori_loop(0, NB, b_body, 0)
    pltpu.sync_copy(log_v, logits.at[pl.ds(b_base, NB)])
    return carry

  lax.fori_loop(0, CHUNKS, chunk_body, 0)


def _sc_logits(in_tbl, out_tbl, labels, ctx_rows):
  mesh = plsc.VectorSubcoreMesh(core_axis_name="c", subcore_axis_name="s")
  kfn = pl.kernel(
      _sc_body,
      out_type=jax.ShapeDtypeStruct((B, LOGIT_COLS), jnp.float32),
      mesh=mesh,
      compiler_params=pltpu.CompilerParams(
          needs_layout_passes=False, use_tc_tiling_on_sc=False),
      scratch_types=[
          pltpu.VMEM((IDX_ROWS, 128), jnp.int32),
          pltpu.VMEM((NB,), jnp.int32),
          pltpu.VMEM((NB * CTX + ROW_PAD, D), jnp.float32),
          pltpu.VMEM((NB, D), jnp.float32),
          pltpu.VMEM((NB, LOGIT_COLS), jnp.float32),
          pltpu.SemaphoreType.DMA,
      ],
  )
  return kfn(in_tbl, out_tbl, labels, ctx_rows)


def _tc_body(m_ref, o_ref):
  x = m_ref[:]
  col = lax.broadcasted_iota(jnp.int32, x.shape, 1)
  sign = jnp.where(col < P, 1.0, -1.0).astype(jnp.float32)
  w = jnp.where(col < P, 1.0 / P,
                jnp.where(col < CTX, 1.0 / N, 0.0)).astype(jnp.float32)
  z = x * sign
  ls = jnp.minimum(z, 0.0) - jnp.log1p(jnp.exp(-jnp.abs(z)))
  o_ref[:] = -jnp.sum(ls * w, axis=1)


def kernel(input_labels, pos_labels, neg_labels, in_embed, out_embed):
  labels = _remap_rows(input_labels.astype(jnp.int32))
  ctx = _remap_rows(jnp.concatenate(
      [pos_labels.astype(jnp.int32), neg_labels.astype(jnp.int32)], axis=1))
  ctx_rows = ctx.reshape(NW * CHUNKS, IDX_ROWS, 128)

  in_tbl = _to_linear_table(in_embed)
  out_tbl = _to_linear_table(out_embed)

  logits = _sc_logits(in_tbl, out_tbl, labels, ctx_rows)

  return pl.pallas_call(
      _tc_body,
      out_shape=jax.ShapeDtypeStruct((B,), jnp.float32),
  )(logits)


# speedup vs baseline: 1.4695x; 1.1257x over previous
import functools

import jax
import jax.numpy as jnp
from jax import lax
from jax.experimental import pallas as pl
from jax.experimental.pallas import tpu as pltpu
from jax.experimental.pallas import tpu_sc as plsc

VOCAB = 1000000
D = 32
B = 16384
P = 10
N = 50
CTX = P + N
LOGIT_COLS = 64

VB = 8192
TBLOCKS = -(-VOCAB // VB)
TROWS = TBLOCKS * VB

NC = 2
NS = 16
NW = NC * NS
B_PER_W = B // NW
NB = 32
CHUNKS = B_PER_W // NB
IDX_ROWS = NB * CTX // 128
ROW_PAD = 4


def _transpose_body(x_ref, o_ref):
  x = x_ref[:]
  xx = jnp.concatenate(
      [x[:, 1024 * k:1024 * (k + 1)] for k in range(8)], axis=0)
  eye = jnp.eye(256, dtype=jnp.float32)
  y = lax.dot_general(xx, eye, (((0,), (0,)), ((), ())),
                      preferred_element_type=jnp.float32)
  o_ref[0:1024, :] = y[:, 0:128]
  o_ref[1024:2048, :] = y[:, 128:256]


def _to_linear_table(table):
  out = pl.pallas_call(
      _transpose_body,
      grid=(TBLOCKS,),
      in_specs=[pl.BlockSpec((D, VB), lambda i: (0, i))],
      out_specs=pl.BlockSpec((2048, 128), lambda i: (i, 0)),
      out_shape=jax.ShapeDtypeStruct((TBLOCKS * 2048, 128), jnp.float32),
  )(table.T)
  return out.reshape(TROWS, D)


def _remap_rows(v):
  return (v & ~8191) | (v & 4096) | ((v & 1023) << 2) | ((v >> 10) & 3)


def _sc_body(in_tbl, out_tbl, labels, ctx_idx, logits, cidx_v, lidx_v,
             rows_v, ctr_v, log_v, sems):
  wid = lax.axis_index("s") * NC + lax.axis_index("c")
  iota16 = lax.iota(jnp.int32, 16)
  col_idx = [jnp.full((16,), d, jnp.int32) for d in range(D)]
  mask3 = iota16 < (CTX - 48)

  def dma_list(buf):
    cps = [pltpu.make_async_copy(in_tbl.at[lidx_v.at[buf]],
                                 ctr_v.at[buf], sems.at[buf])]
    for j in range(IDX_ROWS):
      cps.append(
          pltpu.make_async_copy(out_tbl.at[cidx_v.at[buf, j]],
                                rows_v.at[buf, pl.ds(j * 128, 128)],
                                sems.at[buf]))
    return cps

  def stage(c, buf):
    blk = wid * CHUNKS + c
    b_base = pl.multiple_of(blk * NB, NB)
    pltpu.sync_copy(ctx_idx.at[blk], cidx_v.at[buf])
    pltpu.sync_copy(labels.at[pl.ds(b_base, NB)], lidx_v.at[buf])
    for cp in dma_list(buf):
      cp.start()

  stage(0, 0)

  def chunk_body(c, carry):
    cur = lax.rem(c, 2)
    nxt = 1 - cur

    @pl.when(c < CHUNKS - 1)
    def _():
      stage(c + 1, nxt)

    for cp in dma_list(cur):
      cp.wait()

    def b_body(bl, inner):
      base_pair = bl * CTX
      row0 = iota16 + base_pair
      row_idx = [row0 + g * 16 for g in range(LOGIT_COLS // 16)]
      accs = [jnp.zeros((16,), jnp.float32) for _ in range(LOGIT_COLS // 16)]
      ctr_half = [ctr_v[cur, bl, pl.ds(0, 16)], ctr_v[cur, bl, pl.ds(16, 16)]]
      buf_idx = jnp.full((16,), cur, jnp.int32)
      for d in range(D):
        s = ctr_half[d // 16][d % 16]
        for g in range(LOGIT_COLS // 16):
          m = mask3 if g == 3 else None
          v = plsc.load_gather(rows_v, [buf_idx, row_idx[g], col_idx[d]],
                               mask=m)
          accs[g] = accs[g] + v * s
      for g in range(LOGIT_COLS // 16):
        log_v[bl, pl.ds(g * 16, 16)] = accs[g]
      return inner

    lax.fori_loop(0, NB, b_body, 0)
    blk = wid * CHUNKS + c
    b_base = pl.multiple_of(blk * NB, NB)
    pltpu.sync_copy(log_v, logits.at[pl.ds(b_base, NB)])
    return carry

  lax.fori_loop(0, CHUNKS, chunk_body, 0)


def _sc_logits(in_tbl, out_tbl, labels, ctx_rows):
  mesh = plsc.VectorSubcoreMesh(core_axis_name="c", subcore_axis_name="s")
  kfn = pl.kernel(
      _sc_body,
      out_type=jax.ShapeDtypeStruct((B, LOGIT_COLS), jnp.float32),
      mesh=mesh,
      compiler_params=pltpu.CompilerParams(
          needs_layout_passes=False, use_tc_tiling_on_sc=False),
      scratch_types=[
          pltpu.VMEM((2, IDX_ROWS, 128), jnp.int32),
          pltpu.VMEM((2, NB), jnp.int32),
          pltpu.VMEM((2, NB * CTX, D), jnp.float32),
          pltpu.VMEM((2, NB, D), jnp.float32),
          pltpu.VMEM((NB, LOGIT_COLS), jnp.float32),
          pltpu.SemaphoreType.DMA((2,)),
      ],
  )
  return kfn(in_tbl, out_tbl, labels, ctx_rows)


def _tc_body(m_ref, o_ref):
  x = m_ref[:]
  col = lax.broadcasted_iota(jnp.int32, x.shape, 1)
  sign = jnp.where(col < P, 1.0, -1.0).astype(jnp.float32)
  w = jnp.where(col < P, 1.0 / P,
                jnp.where(col < CTX, 1.0 / N, 0.0)).astype(jnp.float32)
  z = x * sign
  ls = jnp.minimum(z, 0.0) - jnp.log1p(jnp.exp(-jnp.abs(z)))
  ls = jnp.where(col < CTX, ls, 0.0)
  o_ref[:] = -jnp.sum(ls * w, axis=1)


def kernel(input_labels, pos_labels, neg_labels, in_embed, out_embed):
  labels = _remap_rows(input_labels.astype(jnp.int32))
  ctx = _remap_rows(jnp.concatenate(
      [pos_labels.astype(jnp.int32), neg_labels.astype(jnp.int32)], axis=1))
  ctx_rows = ctx.reshape(NW * CHUNKS, IDX_ROWS, 128)

  in_tbl = _to_linear_table(in_embed)
  out_tbl = _to_linear_table(out_embed)

  logits = _sc_logits(in_tbl, out_tbl, labels, ctx_rows)

  return pl.pallas_call(
      _tc_body,
      out_shape=jax.ShapeDtypeStruct((B,), jnp.float32),
  )(logits)


# speedup vs baseline: 1.5959x; 1.0860x over previous
import functools

import jax
import jax.numpy as jnp
from jax import lax
from jax.experimental import pallas as pl
from jax.experimental.pallas import tpu as pltpu
from jax.experimental.pallas import tpu_sc as plsc

VOCAB = 1000000
D = 32
B = 16384
P = 10
N = 50
CTX = P + N
LOGIT_COLS = 64

VB = 16384
TBLOCKS = -(-VOCAB // VB)
TROWS = TBLOCKS * VB

NC = 2
NS = 16
NW = NC * NS
B_PER_W = B // NW
NB = 32
CHUNKS = B_PER_W // NB
IDX_ROWS = NB * CTX // 128
ROW_PAD = 4


def _transpose_body(x_ref, o_ref):
  x = x_ref[:]
  xx = jnp.concatenate(
      [x[:, 1024 * k:1024 * (k + 1)] for k in range(16)], axis=0)
  eye = jnp.eye(512, dtype=jnp.float32)
  y = lax.dot_general(xx, eye, (((0,), (0,)), ((), ())),
                      preferred_element_type=jnp.float32)
  for h in range(4):
    o_ref[1024 * h:1024 * (h + 1), :] = y[:, 128 * h:128 * (h + 1)]


def _to_linear_table(table):
  out = pl.pallas_call(
      _transpose_body,
      grid=(TBLOCKS,),
      in_specs=[pl.BlockSpec((D, VB), lambda i: (0, i))],
      out_specs=pl.BlockSpec((4096, 128), lambda i: (i, 0)),
      out_shape=jax.ShapeDtypeStruct((TBLOCKS * 4096, 128), jnp.float32),
  )(table.T)
  return out.reshape(TROWS, D)


def _remap_rows(v):
  return (v & ~4095) | ((v & 1023) << 2) | ((v >> 10) & 3)


def _sc_body(in_tbl, out_tbl, labels, ctx_idx, logits, cidx_v, lidx_v,
             rows_v, ctr_v, log_v, sems):
  wid = lax.axis_index("s") * NC + lax.axis_index("c")
  iota16 = lax.iota(jnp.int32, 16)
  col_idx = [jnp.full((16,), d, jnp.int32) for d in range(D)]
  mask3 = iota16 < (CTX - 48)

  def dma_list(buf):
    cps = [pltpu.make_async_copy(in_tbl.at[lidx_v.at[buf]],
                                 ctr_v.at[buf], sems.at[buf])]
    for j in range(IDX_ROWS):
      cps.append(
          pltpu.make_async_copy(out_tbl.at[cidx_v.at[buf, j]],
                                rows_v.at[buf, pl.ds(j * 128, 128)],
                                sems.at[buf]))
    return cps

  def stage(c, buf):
    blk = wid * CHUNKS + c
    b_base = pl.multiple_of(blk * NB, NB)
    pltpu.sync_copy(ctx_idx.at[blk], cidx_v.at[buf])
    pltpu.sync_copy(labels.at[pl.ds(b_base, NB)], lidx_v.at[buf])
    for cp in dma_list(buf):
      cp.start()

  stage(0, 0)

  def chunk_body(c, carry):
    cur = lax.rem(c, 2)
    nxt = 1 - cur

    @pl.when(c < CHUNKS - 1)
    def _():
      stage(c + 1, nxt)

    for cp in dma_list(cur):
      cp.wait()

    def b_body(bl, inner):
      base_pair = bl * CTX
      row0 = iota16 + base_pair
      row_idx = [row0 + g * 16 for g in range(LOGIT_COLS // 16)]
      accs = [jnp.zeros((16,), jnp.float32) for _ in range(LOGIT_COLS // 16)]
      ctr_half = [ctr_v[cur, bl, pl.ds(0, 16)], ctr_v[cur, bl, pl.ds(16, 16)]]
      buf_idx = jnp.full((16,), cur, jnp.int32)
      for d in range(D):
        s = ctr_half[d // 16][d % 16]
        for g in range(LOGIT_COLS // 16):
          m = mask3 if g == 3 else None
          v = plsc.load_gather(rows_v, [buf_idx, row_idx[g], col_idx[d]],
                               mask=m)
          accs[g] = accs[g] + v * s
      for g in range(LOGIT_COLS // 16):
        log_v[bl, pl.ds(g * 16, 16)] = accs[g]
      return inner

    lax.fori_loop(0, NB, b_body, 0)
    blk = wid * CHUNKS + c
    b_base = pl.multiple_of(blk * NB, NB)
    pltpu.sync_copy(log_v, logits.at[pl.ds(b_base, NB)])
    return carry

  lax.fori_loop(0, CHUNKS, chunk_body, 0)


def _sc_logits(in_tbl, out_tbl, labels, ctx_rows):
  mesh = plsc.VectorSubcoreMesh(core_axis_name="c", subcore_axis_name="s")
  kfn = pl.kernel(
      _sc_body,
      out_type=jax.ShapeDtypeStruct((B, LOGIT_COLS), jnp.float32),
      mesh=mesh,
      compiler_params=pltpu.CompilerParams(
          needs_layout_passes=False, use_tc_tiling_on_sc=False),
      scratch_types=[
          pltpu.VMEM((2, IDX_ROWS, 128), jnp.int32),
          pltpu.VMEM((2, NB), jnp.int32),
          pltpu.VMEM((2, NB * CTX, D), jnp.float32),
          pltpu.VMEM((2, NB, D), jnp.float32),
          pltpu.VMEM((NB, LOGIT_COLS), jnp.float32),
          pltpu.SemaphoreType.DMA((2,)),
      ],
  )
  return kfn(in_tbl, out_tbl, labels, ctx_rows)


def _tc_body(m_ref, o_ref):
  x = m_ref[:]
  col = lax.broadcasted_iota(jnp.int32, x.shape, 1)
  sign = jnp.where(col < P, 1.0, -1.0).astype(jnp.float32)
  w = jnp.where(col < P, 1.0 / P,
                jnp.where(col < CTX, 1.0 / N, 0.0)).astype(jnp.float32)
  z = x * sign
  ls = jnp.minimum(z, 0.0) - jnp.log1p(jnp.exp(-jnp.abs(z)))
  ls = jnp.where(col < CTX, ls, 0.0)
  o_ref[:] = -jnp.sum(ls * w, axis=1)


def kernel(input_labels, pos_labels, neg_labels, in_embed, out_embed):
  labels = _remap_rows(input_labels.astype(jnp.int32))
  ctx = _remap_rows(jnp.concatenate(
      [pos_labels.astype(jnp.int32), neg_labels.astype(jnp.int32)], axis=1))
  ctx_rows = ctx.reshape(NW * CHUNKS, IDX_ROWS, 128)

  in_tbl = _to_linear_table(in_embed)
  out_tbl = _to_linear_table(out_embed)

  logits = _sc_logits(in_tbl, out_tbl, labels, ctx_rows)

  return pl.pallas_call(
      _tc_body,
      out_shape=jax.ShapeDtypeStruct((B,), jnp.float32),
  )(logits)


# speedup vs baseline: 1.5974x; 1.0010x over previous
import functools

import jax
import jax.numpy as jnp
from jax import lax
from jax.experimental import pallas as pl
from jax.experimental.pallas import tpu as pltpu
from jax.experimental.pallas import tpu_sc as plsc

VOCAB = 1000000
D = 32
B = 16384
P = 10
N = 50
CTX = P + N
LOGIT_COLS = 64

VB = 16384
TBLOCKS = -(-VOCAB // VB)
TROWS = TBLOCKS * VB

NC = 2
NS = 16
NW = NC * NS
B_PER_W = B // NW
NB = 32
CHUNKS = B_PER_W // NB
IDX_ROWS = NB * CTX // 128
ROW_PAD = 4


def _transpose_body(x_ref, o_ref):
  x = x_ref[:]
  xx = jnp.concatenate(
      [x[:, 1024 * k:1024 * (k + 1)] for k in range(16)], axis=0)
  eye = jnp.eye(512, dtype=jnp.float32)
  y = lax.dot_general(xx, eye, (((0,), (0,)), ((), ())),
                      preferred_element_type=jnp.float32)
  for h in range(4):
    o_ref[1024 * h:1024 * (h + 1), :] = y[:, 128 * h:128 * (h + 1)]


def _to_linear_table(table):
  out = pl.pallas_call(
      _transpose_body,
      grid=(TBLOCKS,),
      in_specs=[pl.BlockSpec((D, VB), lambda i: (0, i))],
      out_specs=pl.BlockSpec((4096, 128), lambda i: (i, 0)),
      out_shape=jax.ShapeDtypeStruct((TBLOCKS * 4096, 128), jnp.float32),
  )(table.T)
  return out.reshape(TROWS, D)


def _remap_rows(v):
  return (v & ~4095) | ((v & 1023) << 2) | ((v >> 10) & 3)


def _sc_body(in_tbl, out_tbl, labels, ctx_idx, logits, cidx_v, lidx_v,
             rows_v, ctr_v, log_v, sems):
  wid = lax.axis_index("s") * NC + lax.axis_index("c")
  iota16 = lax.iota(jnp.int32, 16)
  col_idx = [jnp.full((16,), d, jnp.int32) for d in range(D)]
  mask3 = iota16 < (CTX - 48)

  def dma_list(buf):
    cps = [pltpu.make_async_copy(in_tbl.at[lidx_v.at[buf]],
                                 ctr_v.at[buf], sems.at[buf])]
    for j in range(IDX_ROWS):
      cps.append(
          pltpu.make_async_copy(out_tbl.at[cidx_v.at[buf, j]],
                                rows_v.at[buf, pl.ds(j * 128, 128)],
                                sems.at[buf]))
    return cps

  def stage(c, buf):
    blk = wid * CHUNKS + c
    b_base = pl.multiple_of(blk * NB, NB)
    row0 = pl.multiple_of(blk * IDX_ROWS, IDX_ROWS)
    pltpu.sync_copy(ctx_idx.at[pl.ds(row0, IDX_ROWS)], cidx_v.at[buf])
    pltpu.sync_copy(labels.at[pl.ds(b_base, NB)], lidx_v.at[buf])
    for cp in dma_list(buf):
      cp.start()

  stage(0, 0)

  def chunk_body(c, carry):
    cur = lax.rem(c, 2)
    nxt = 1 - cur

    @pl.when(c < CHUNKS - 1)
    def _():
      stage(c + 1, nxt)

    for cp in dma_list(cur):
      cp.wait()

    def b_body(bl, inner):
      base_pair = bl * CTX
      row0 = iota16 + base_pair
      row_idx = [row0 + g * 16 for g in range(LOGIT_COLS // 16)]
      accs = [jnp.zeros((16,), jnp.float32) for _ in range(LOGIT_COLS // 16)]
      ctr_half = [ctr_v[cur, bl, pl.ds(0, 16)], ctr_v[cur, bl, pl.ds(16, 16)]]
      buf_idx = jnp.full((16,), cur, jnp.int32)
      for d in range(D):
        s = ctr_half[d // 16][d % 16]
        for g in range(LOGIT_COLS // 16):
          m = mask3 if g == 3 else None
          v = plsc.load_gather(rows_v, [buf_idx, row_idx[g], col_idx[d]],
                               mask=m)
          accs[g] = accs[g] + v * s
      for g in range(LOGIT_COLS // 16):
        log_v[bl, pl.ds(g * 16, 16)] = accs[g]
      return inner

    lax.fori_loop(0, NB, b_body, 0)
    blk = wid * CHUNKS + c
    b_base = pl.multiple_of(blk * NB, NB)
    pltpu.sync_copy(log_v, logits.at[pl.ds(b_base, NB)])
    return carry

  lax.fori_loop(0, CHUNKS, chunk_body, 0)


def _sc_logits(in_tbl, out_tbl, labels, ctx_rows):
  mesh = plsc.VectorSubcoreMesh(core_axis_name="c", subcore_axis_name="s")
  kfn = pl.kernel(
      _sc_body,
      out_type=jax.ShapeDtypeStruct((B, LOGIT_COLS), jnp.float32),
      mesh=mesh,
      compiler_params=pltpu.CompilerParams(
          needs_layout_passes=False, use_tc_tiling_on_sc=False),
      scratch_types=[
          pltpu.VMEM((2, IDX_ROWS, 128), jnp.int32),
          pltpu.VMEM((2, NB), jnp.int32),
          pltpu.VMEM((2, NB * CTX, D), jnp.float32),
          pltpu.VMEM((2, NB, D), jnp.float32),
          pltpu.VMEM((NB, LOGIT_COLS), jnp.float32),
          pltpu.SemaphoreType.DMA((2,)),
      ],
  )
  return kfn(in_tbl, out_tbl, labels, ctx_rows)


def _tc_body(m_ref, o_ref):
  x = m_ref[:]
  col = lax.broadcasted_iota(jnp.int32, x.shape, 1)
  sign = jnp.where(col < P, 1.0, -1.0).astype(jnp.float32)
  w = jnp.where(col < P, 1.0 / P,
                jnp.where(col < CTX, 1.0 / N, 0.0)).astype(jnp.float32)
  z = x * sign
  ls = jnp.minimum(z, 0.0) - jnp.log1p(jnp.exp(-jnp.abs(z)))
  ls = jnp.where(col < CTX, ls, 0.0)
  o_ref[:] = -jnp.sum(ls * w, axis=1)


def kernel(input_labels, pos_labels, neg_labels, in_embed, out_embed):
  labels = _remap_rows(input_labels.astype(jnp.int32))
  ctx = _remap_rows(jnp.concatenate(
      [pos_labels.astype(jnp.int32), neg_labels.astype(jnp.int32)], axis=1))
  ctx_rows = ctx.reshape(B * CTX // 128, 128)

  in_tbl = _to_linear_table(in_embed)
  out_tbl = _to_linear_table(out_embed)

  logits = _sc_logits(in_tbl, out_tbl, labels, ctx_rows)

  return pl.pallas_call(
      _tc_body,
      out_shape=jax.ShapeDtypeStruct((B,), jnp.float32),
  )(logits)
